# baseline (speedup 1.0000x reference)
import jax
import jax.numpy as jnp
from jax import lax
from jax.experimental import pallas as pl
from jax.experimental.pallas import tpu as pltpu
from jax.experimental.pallas import tpu_sc as plsc

D = 64
OW = 128
V = 1000000
B = 1024 * 200
SCALE = 8.0
LANES = 16
NUM_CORES = 2
NW = 32

RANGE = V // NW
SLABW = 512
NSLAB = 62
PC = 12272
XC = 4096
NCHUNK = B // XC
NVREG = B // LANES
VPC = XC // LANES


def _body(x_hbm, tT_hbm, tail_hbm, out_hbm,
          xbuf, vlA, blA, vlB, blB, slab, rowbuf, tailbuf,
          st_a, ct_a, st_b, ct_b,
          sem_a, sem_b, sem_o):
    w = lax.axis_index("s") * NUM_CORES + lax.axis_index("c")
    lo = w * RANGE
    alo = (lo >> 7) << 7
    tail512 = alo + (NSLAB - 1) * SLABW
    tail_ok = tail512 + SLABW <= V
    iota = lax.iota(jnp.int32, LANES)

    def col0_of(u):
        tb = jnp.where(tail_ok, tail512, V - D)
        c0 = jnp.where(u == NSLAB - 1, tb, alo + u * SLABW)
        return pl.multiple_of(c0, 128)

    def is_short_slab(u):
        return jnp.logical_and(u == NSLAB - 1, jnp.logical_not(tail_ok))

    def fire_slab(u, srow, sem):
        c0 = col0_of(u)

        @pl.when(jnp.logical_not(is_short_slab(u)))
        def _():
            for tr in range(D // 8):
                pltpu.async_copy(
                    tT_hbm.at[pl.ds(tr * 8, 8), pl.ds(c0, SLABW)],
                    slab.at[pl.ds(srow + tr * 8, 8)], sem)

    def drain_slab(u, srow, sem):
        @pl.when(jnp.logical_not(is_short_slab(u)))
        def _():
            pltpu.make_async_copy(
                tT_hbm.at[pl.ds(0, D), pl.ds(0, SLABW)],
                slab.at[pl.ds(srow, D)], sem).wait()

    def drain_out_one():
        pltpu.make_async_copy(
            tT_hbm.at[pl.ds(0, LANES), pl.ds(0, OW)],
            rowbuf.at[pl.ds(0, LANES)], sem_o).wait()

    pltpu.sync_copy(tail_hbm, tailbuf)

    def round_body(s0):
        fire_slab(0, 0, sem_a)
        fire_slab(1, D, sem_b)

        def scan_chunk(c, carry):
            pltpu.sync_copy(x_hbm.at[pl.ds(c * XC, XC)], xbuf)

            def scan_vreg(j, carry2):
                off, collecting, cutoff = carry2
                k = c * VPC + j
                v16 = xbuf[pl.ds(j * LANES, LANES)]
                m = jnp.logical_and(v16 >= lo, v16 < lo + RANGE)
                cnt = plsc.all_reduce_population_count(m)[0]
                active = k >= s0
                fits = off + cnt <= PC
                do = jnp.logical_and(jnp.logical_and(active, collecting), fits)
                mm = jnp.logical_and(m, do)
                plsc.store_compressed(vlA.at[pl.ds(off, LANES)], v16, mask=mm)
                b16 = iota + k * LANES
                plsc.store_compressed(blA.at[pl.ds(off, LANES)], b16, mask=mm)
                off = off + jnp.where(do, cnt, 0)
                stop = jnp.logical_and(jnp.logical_and(active, collecting),
                                       jnp.logical_not(fits))
                cutoff = jnp.where(stop, k, cutoff)
                collecting = jnp.logical_and(collecting, jnp.logical_not(stop))
                return (off, collecting, cutoff)

            return lax.fori_loop(0, VPC, scan_vreg, carry)

        npairs, _, cutoff = lax.fori_loop(
            0, NCHUNK, scan_chunk,
            (jnp.int32(0), jnp.bool_(True), jnp.int32(NVREG)))

        st_a[0] = jnp.int32(0)
        ct_a[0] = npairs
        srcs = [(vlA, blA, st_a, ct_a), (vlB, blB, st_b, ct_b)]
        for lvl in range(6):
            vs, bs, sts, cts = srcs[lvl % 2]
            vd, bd, std, ctd = srcs[(lvl + 1) % 2]
            bit = 14 - lvl

            def split_list(i, _, vs=vs, bs=bs, sts=sts, cts=cts,
                           vd=vd, bd=bd, std=std, ctd=ctd, bit=bit):
                s = sts[i]
                c = cts[i]
                nv = (c + LANES - 1) >> 4

                def count_vreg(k, nlo):
                    v16 = vs[pl.ds(s + k * LANES, LANES)]
                    lanem = iota < (c - k * LANES)
                    bitm = ((v16 - alo) >> bit) & 1
                    mlo = jnp.logical_and(lanem, bitm == 0)
                    return nlo + plsc.all_reduce_population_count(mlo)[0]

                nlo = lax.fori_loop(0, nv, count_vreg, jnp.int32(0))

                def place_vreg(k, offs):
                    lo_off, hi_off = offs
                    v16 = vs[pl.ds(s + k * LANES, LANES)]
                    b16 = bs[pl.ds(s + k * LANES, LANES)]
                    lanem = iota < (c - k * LANES)
                    bitm = ((v16 - alo) >> bit) & 1
                    mlo = jnp.logical_and(lanem, bitm == 0)
                    mhi = jnp.logical_and(lanem, bitm == 1)
                    plsc.store_compressed(vd.at[pl.ds(lo_off, LANES)], v16, mask=mlo)
                    plsc.store_compressed(bd.at[pl.ds(lo_off, LANES)], b16, mask=mlo)
                    plsc.store_compressed(vd.at[pl.ds(hi_off, LANES)], v16, mask=mhi)
                    plsc.store_compressed(bd.at[pl.ds(hi_off, LANES)], b16, mask=mhi)
                    clo = plsc.all_reduce_population_count(mlo)[0]
                    chi = plsc.all_reduce_population_count(mhi)[0]
                    return (lo_off + clo, hi_off + chi)

                lax.fori_loop(0, nv, place_vreg, (s, s + nlo))
                std[2 * i] = s
                ctd[2 * i] = nlo
                std[2 * i + 1] = s + nlo
                ctd[2 * i + 1] = c - nlo
                return 0

            lax.fori_loop(0, 1 << lvl, split_list, 0)

        def extract_from(u, srow, fired, tail):
            s = st_a[u]
            c = ct_a[u]
            c0 = col0_of(u)
            nv = (c + LANES - 1) >> 4

            def pairs_vreg(k, fired2):
                v16 = vlA[pl.ds(s + k * LANES, LANES)]
                b16 = blA[pl.ds(s + k * LANES, LANES)]
                c16 = v16 - c0
                sc16 = jnp.where(v16 == 0, 0.0, SCALE).astype(jnp.float32)
                rem = c - k * LANES
                lanem = iota < rem
                bsafe = jnp.where(lanem, b16, B)
                lax.fori_loop(0, jnp.where(fired2 >= 2, 1, 0),
                              lambda q, c3: (drain_out_one(), c3)[1], 0)
                rb = (fired2 & 1) * LANES
                for r in range(LANES):
                    @pl.when(rem > r)
                    def _():
                        cc = c16[r]
                        ss = sc16[r]
                        ccv = jnp.full((LANES,), cc, jnp.int32)
                        for g in range(D // LANES):
                            dv = iota + (srow + g * LANES)
                            if tail:
                                g16 = plsc.load_gather(tailbuf, [ccv, dv - srow])
                            else:
                                g16 = plsc.load_gather(slab, [dv, ccv])
                            rowbuf[rb + r, pl.ds(g * LANES, LANES)] = g16 * ss
                pltpu.async_copy(rowbuf.at[pl.ds(rb, LANES)],
                                 out_hbm.at[bsafe], sem_o)
                return fired2 + 1

            return lax.fori_loop(0, nv, pairs_vreg, fired)

        def extract(u, srow, fired):
            return lax.cond(
                is_short_slab(u),
                lambda f: extract_from(u, srow, f, True),
                lambda f: extract_from(u, srow, f, False),
                fired)

        def sweep_pair(t, fired):
            u0 = 2 * t
            drain_slab(u0, 0, sem_a)
            fired = extract(u0, 0, fired)

            @pl.when(u0 + 2 < NSLAB)
            def _():
                fire_slab(u0 + 2, 0, sem_a)
            drain_slab(u0 + 1, D, sem_b)
            fired = extract(u0 + 1, D, fired)

            @pl.when(u0 + 3 < NSLAB)
            def _():
                fire_slab(u0 + 3, D, sem_b)
            return fired

        fired = lax.fori_loop(0, NSLAB // 2, sweep_pair, jnp.int32(0))
        lax.fori_loop(0, jnp.minimum(fired, 2),
                      lambda q, c3: (drain_out_one(), c3)[1], 0)
        return cutoff

    lax.while_loop(lambda s0: s0 < NVREG, round_body, jnp.int32(0))


def kernel(x, table):
    x1 = x.reshape(B).astype(jnp.int32)
    tT = table.T
    tail = jnp.pad(table[V - D:, :], ((0, 0), (0, OW - D)))
    mesh = plsc.VectorSubcoreMesh(core_axis_name="c", subcore_axis_name="s")
    out = pl.kernel(
        _body,
        mesh=mesh,
        compiler_params=pltpu.CompilerParams(needs_layout_passes=False),
        out_type=jax.ShapeDtypeStruct((B + LANES, OW), jnp.float32),
        scratch_types=[
            pltpu.VMEM((XC,), jnp.int32),
            pltpu.VMEM((PC,), jnp.int32),
            pltpu.VMEM((PC,), jnp.int32),
            pltpu.VMEM((PC,), jnp.int32),
            pltpu.VMEM((PC,), jnp.int32),
            pltpu.VMEM((2 * D, SLABW), jnp.float32),
            pltpu.VMEM((2 * LANES, OW), jnp.float32),
            pltpu.VMEM((D, OW), jnp.float32),
            pltpu.SMEM((64,), jnp.int32),
            pltpu.SMEM((64,), jnp.int32),
            pltpu.SMEM((64,), jnp.int32),
            pltpu.SMEM((64,), jnp.int32),
            pltpu.SemaphoreType.DMA,
            pltpu.SemaphoreType.DMA,
            pltpu.SemaphoreType.DMA,
        ],
    )(x1, tT, tail)
    return out[:B, :D].reshape(x.shape[0], x.shape[1], D)


# speedup vs baseline: 1.0046x; 1.0046x over previous
import jax
import jax.numpy as jnp
from jax import lax
from jax.experimental import pallas as pl
from jax.experimental.pallas import tpu as pltpu
from jax.experimental.pallas import tpu_sc as plsc

D = 64
OW = 128
V = 1000000
B = 1024 * 200
SCALE = 8.0
LANES = 16
NUM_CORES = 2
NW = 32

RANGE = V // NW
SLABW = 512
NSLAB = 62
PC = 9200
RING = 8
XC = 4096
NCHUNK = B // XC
NVREG = B // LANES
VPC = XC // LANES


def _body(x_hbm, tT_hbm, tail_hbm, out_hbm,
          xbuf, vlA, blA, vlB, blB, slab, rowbuf, tailbuf,
          st_a, ct_a, st_b, ct_b,
          sem_a, sem_b, sem_o):
    w = lax.axis_index("s") * NUM_CORES + lax.axis_index("c")
    lo = w * RANGE
    alo = (lo >> 7) << 7
    tail512 = alo + (NSLAB - 1) * SLABW
    tail_ok = tail512 + SLABW <= V
    iota = lax.iota(jnp.int32, LANES)

    def col0_of(u):
        tb = jnp.where(tail_ok, tail512, V - D)
        c0 = jnp.where(u == NSLAB - 1, tb, alo + u * SLABW)
        return pl.multiple_of(c0, 128)

    def is_short_slab(u):
        return jnp.logical_and(u == NSLAB - 1, jnp.logical_not(tail_ok))

    def fire_slab(u, srow, sem):
        c0 = col0_of(u)

        @pl.when(jnp.logical_not(is_short_slab(u)))
        def _():
            for tr in range(D // 8):
                pltpu.async_copy(
                    tT_hbm.at[pl.ds(tr * 8, 8), pl.ds(c0, SLABW)],
                    slab.at[pl.ds(srow + tr * 8, 8)], sem)

    def drain_slab(u, srow, sem):
        @pl.when(jnp.logical_not(is_short_slab(u)))
        def _():
            pltpu.make_async_copy(
                tT_hbm.at[pl.ds(0, D), pl.ds(0, SLABW)],
                slab.at[pl.ds(srow, D)], sem).wait()

    def drain_out_one():
        pltpu.make_async_copy(
            tT_hbm.at[pl.ds(0, LANES), pl.ds(0, OW)],
            rowbuf.at[pl.ds(0, LANES)], sem_o).wait()

    pltpu.sync_copy(tail_hbm, tailbuf)

    def round_body(s0):
        fire_slab(0, 0, sem_a)
        fire_slab(1, D, sem_b)

        def scan_chunk(c, carry):
            pltpu.sync_copy(x_hbm.at[pl.ds(c * XC, XC)], xbuf)

            def scan_vreg(j, carry2):
                off, collecting, cutoff = carry2
                k = c * VPC + j
                v16 = xbuf[pl.ds(j * LANES, LANES)]
                m = jnp.logical_and(v16 >= lo, v16 < lo + RANGE)
                cnt = plsc.all_reduce_population_count(m)[0]
                active = k >= s0
                fits = off + cnt <= PC
                do = jnp.logical_and(jnp.logical_and(active, collecting), fits)
                mm = jnp.logical_and(m, do)
                plsc.store_compressed(vlA.at[pl.ds(off, LANES)], v16, mask=mm)
                b16 = iota + k * LANES
                plsc.store_compressed(blA.at[pl.ds(off, LANES)], b16, mask=mm)
                off = off + jnp.where(do, cnt, 0)
                stop = jnp.logical_and(jnp.logical_and(active, collecting),
                                       jnp.logical_not(fits))
                cutoff = jnp.where(stop, k, cutoff)
                collecting = jnp.logical_and(collecting, jnp.logical_not(stop))
                return (off, collecting, cutoff)

            return lax.fori_loop(0, VPC, scan_vreg, carry)

        npairs, _, cutoff = lax.fori_loop(
            0, NCHUNK, scan_chunk,
            (jnp.int32(0), jnp.bool_(True), jnp.int32(NVREG)))

        st_a[0] = jnp.int32(0)
        ct_a[0] = npairs
        srcs = [(vlA, blA, st_a, ct_a), (vlB, blB, st_b, ct_b)]
        for lvl in range(6):
            vs, bs, sts, cts = srcs[lvl % 2]
            vd, bd, std, ctd = srcs[(lvl + 1) % 2]
            bit = 14 - lvl

            def split_list(i, _, vs=vs, bs=bs, sts=sts, cts=cts,
                           vd=vd, bd=bd, std=std, ctd=ctd, bit=bit):
                s = sts[i]
                c = cts[i]
                nv = (c + LANES - 1) >> 4

                def count_vreg(k, nlo):
                    v16 = vs[pl.ds(s + k * LANES, LANES)]
                    lanem = iota < (c - k * LANES)
                    bitm = ((v16 - alo) >> bit) & 1
                    mlo = jnp.logical_and(lanem, bitm == 0)
                    return nlo + plsc.all_reduce_population_count(mlo)[0]

                nlo = lax.fori_loop(0, nv, count_vreg, jnp.int32(0))

                def place_vreg(k, offs):
                    lo_off, hi_off = offs
                    v16 = vs[pl.ds(s + k * LANES, LANES)]
                    b16 = bs[pl.ds(s + k * LANES, LANES)]
                    lanem = iota < (c - k * LANES)
                    bitm = ((v16 - alo) >> bit) & 1
                    mlo = jnp.logical_and(lanem, bitm == 0)
                    mhi = jnp.logical_and(lanem, bitm == 1)
                    plsc.store_compressed(vd.at[pl.ds(lo_off, LANES)], v16, mask=mlo)
                    plsc.store_compressed(bd.at[pl.ds(lo_off, LANES)], b16, mask=mlo)
                    plsc.store_compressed(vd.at[pl.ds(hi_off, LANES)], v16, mask=mhi)
                    plsc.store_compressed(bd.at[pl.ds(hi_off, LANES)], b16, mask=mhi)
                    clo = plsc.all_reduce_population_count(mlo)[0]
                    chi = plsc.all_reduce_population_count(mhi)[0]
                    return (lo_off + clo, hi_off + chi)

                lax.fori_loop(0, nv, place_vreg, (s, s + nlo))
                std[2 * i] = s
                ctd[2 * i] = nlo
                std[2 * i + 1] = s + nlo
                ctd[2 * i + 1] = c - nlo
                return 0

            lax.fori_loop(0, 1 << lvl, split_list, 0)

        def extract_from(u, srow, fired, tail):
            s = st_a[u]
            c = ct_a[u]
            c0 = col0_of(u)
            nv = (c + LANES - 1) >> 4

            def pairs_vreg(k, fired2):
                v16 = vlA[pl.ds(s + k * LANES, LANES)]
                b16 = blA[pl.ds(s + k * LANES, LANES)]
                cmax = D - 1 if tail else SLABW - 1
                c16 = jnp.clip(v16 - c0, 0, cmax)
                sc16 = jnp.where(v16 == 0, 0.0, SCALE).astype(jnp.float32)
                rem = c - k * LANES
                lanem = iota < rem
                bsafe = jnp.where(lanem, b16, B)

                @pl.when(fired2 >= RING)
                def _():
                    drain_out_one()
                rb = (fired2 & (RING - 1)) * LANES
                for r in range(LANES):
                    cc = c16[r]
                    ss = sc16[r]
                    ccv = jnp.full((LANES,), cc, jnp.int32)
                    for g in range(D // LANES):
                        dv = iota + (srow + g * LANES)
                        if tail:
                            g16 = plsc.load_gather(tailbuf, [ccv, dv - srow])
                        else:
                            g16 = plsc.load_gather(slab, [dv, ccv])
                        rowbuf[rb + r, pl.ds(g * LANES, LANES)] = g16 * ss
                pltpu.async_copy(rowbuf.at[pl.ds(rb, LANES)],
                                 out_hbm.at[bsafe], sem_o)
                return fired2 + 1

            return lax.fori_loop(0, nv, pairs_vreg, fired)

        def extract(u, srow, fired):
            return lax.cond(
                is_short_slab(u),
                lambda f: extract_from(u, srow, f, True),
                lambda f: extract_from(u, srow, f, False),
                fired)

        def sweep_pair(t, fired):
            u0 = 2 * t
            drain_slab(u0, 0, sem_a)
            fired = extract(u0, 0, fired)

            @pl.when(u0 + 2 < NSLAB)
            def _():
                fire_slab(u0 + 2, 0, sem_a)
            drain_slab(u0 + 1, D, sem_b)
            fired = extract(u0 + 1, D, fired)

            @pl.when(u0 + 3 < NSLAB)
            def _():
                fire_slab(u0 + 3, D, sem_b)
            return fired

        fired = lax.fori_loop(0, NSLAB // 2, sweep_pair, jnp.int32(0))
        lax.fori_loop(0, jnp.minimum(fired, RING),
                      lambda q, c3: (drain_out_one(), c3)[1], 0)
        return cutoff

    lax.while_loop(lambda s0: s0 < NVREG, round_body, jnp.int32(0))


def kernel(x, table):
    x1 = x.reshape(B).astype(jnp.int32)
    tT = table.T
    tail = jnp.pad(table[V - D:, :], ((0, 0), (0, OW - D)))
    mesh = plsc.VectorSubcoreMesh(core_axis_name="c", subcore_axis_name="s")
    out = pl.kernel(
        _body,
        mesh=mesh,
        compiler_params=pltpu.CompilerParams(needs_layout_passes=False),
        out_type=jax.ShapeDtypeStruct((B + LANES, OW), jnp.float32),
        scratch_types=[
            pltpu.VMEM((XC,), jnp.int32),
            pltpu.VMEM((PC,), jnp.int32),
            pltpu.VMEM((PC,), jnp.int32),
            pltpu.VMEM((PC,), jnp.int32),
            pltpu.VMEM((PC,), jnp.int32),
            pltpu.VMEM((2 * D, SLABW), jnp.float32),
            pltpu.VMEM((RING * LANES, OW), jnp.float32),
            pltpu.VMEM((D, OW), jnp.float32),
            pltpu.SMEM((64,), jnp.int32),
            pltpu.SMEM((64,), jnp.int32),
            pltpu.SMEM((64,), jnp.int32),
            pltpu.SMEM((64,), jnp.int32),
            pltpu.SemaphoreType.DMA,
            pltpu.SemaphoreType.DMA,
            pltpu.SemaphoreType.DMA,
        ],
    )(x1, tT, tail)
    return out[:B, :D].reshape(x.shape[0], x.shape[1], D)


# speedup vs baseline: 1.1057x; 1.1007x over previous
import jax
import jax.numpy as jnp
from jax import lax
from jax.experimental import pallas as pl
from jax.experimental.pallas import tpu as pltpu
from jax.experimental.pallas import tpu_sc as plsc

_DN = lax.GatherDimensionNumbers(
    offset_dims=(), collapsed_slice_dims=(0,), start_index_map=(0,))


def _bcast(vec, lane):
    idx = jnp.full((16, 1), lane, jnp.int32)
    return lax.gather(vec, idx, _DN, (1,),
                      mode=lax.GatherScatterMode.PROMISE_IN_BOUNDS)

D = 64
OW = 128
V = 1000000
B = 1024 * 200
SCALE = 8.0
LANES = 16
NUM_CORES = 2
NW = 32

RANGE = V // NW
SLABW = 512
NSLAB = 62
PC = 8176
RING = 8
XC = 4096
NCHUNK = B // XC
NVREG = B // LANES
VPC = XC // LANES


def _body(x_hbm, tT_hbm, tail_hbm, out_hbm,
          xbuf, vlA, blA, vlB, blB, slab, rowbuf, tailbuf,
          st_a, ct_a, st_b, ct_b,
          sem_a, sem_b, sem_o, sem_x):
    w = lax.axis_index("s") * NUM_CORES + lax.axis_index("c")
    lo = w * RANGE
    alo = (lo >> 7) << 7
    tail512 = alo + (NSLAB - 1) * SLABW
    tail_ok = tail512 + SLABW <= V
    iota = lax.iota(jnp.int32, LANES)

    def col0_of(u):
        tb = jnp.where(tail_ok, tail512, V - D)
        c0 = jnp.where(u == NSLAB - 1, tb, alo + u * SLABW)
        return pl.multiple_of(c0, 128)

    def is_short_slab(u):
        return jnp.logical_and(u == NSLAB - 1, jnp.logical_not(tail_ok))

    def fire_slab(u, srow, sem):
        c0 = col0_of(u)

        @pl.when(jnp.logical_not(is_short_slab(u)))
        def _():
            for tr in range(D // 8):
                pltpu.async_copy(
                    tT_hbm.at[pl.ds(tr * 8, 8), pl.ds(c0, SLABW)],
                    slab.at[pl.ds(srow + tr * 8, 8)], sem)

    def drain_slab(u, srow, sem):
        @pl.when(jnp.logical_not(is_short_slab(u)))
        def _():
            pltpu.make_async_copy(
                tT_hbm.at[pl.ds(0, D), pl.ds(0, SLABW)],
                slab.at[pl.ds(srow, D)], sem).wait()

    def drain_out_one():
        pltpu.make_async_copy(
            tT_hbm.at[pl.ds(0, LANES), pl.ds(0, OW)],
            rowbuf.at[pl.ds(0, LANES)], sem_o).wait()

    pltpu.sync_copy(tail_hbm, tailbuf)

    def round_body(s0):
        fire_slab(0, 0, sem_a)
        fire_slab(1, D, sem_b)

        pltpu.async_copy(x_hbm.at[pl.ds(0, XC)], xbuf.at[pl.ds(0, XC)], sem_x)

        def scan_chunk(c, carry):
            pltpu.make_async_copy(x_hbm.at[pl.ds(0, XC)],
                                  xbuf.at[pl.ds(0, XC)], sem_x).wait()

            @pl.when(c + 1 < NCHUNK)
            def _():
                pltpu.async_copy(
                    x_hbm.at[pl.ds((c + 1) * XC, XC)],
                    xbuf.at[pl.ds(((c + 1) & 1) * XC, XC)], sem_x)
            xb = (c & 1) * XC

            def scan_block(bi, carry2):
                base16, collecting, cutoff = carry2
                k0 = c * VPC + bi * 16
                bb0 = base16[0]
                fits = bb0 <= PC - 256
                fresh = jnp.logical_and(collecting, k0 >= s0)
                ok = jnp.logical_and(fresh, fits)
                stop = jnp.logical_and(fresh, jnp.logical_not(fits))
                cutoff = jnp.where(stop, k0, cutoff)
                collecting = jnp.logical_and(collecting,
                                             jnp.logical_not(stop))

                def scan_vreg(j2, b16c):
                    j = bi * 16 + j2
                    v16 = xbuf[pl.ds(xb + j * LANES, LANES)]
                    m = jnp.logical_and(
                        jnp.logical_and(v16 >= lo, v16 < lo + RANGE), ok)
                    mi = m.astype(jnp.int32)
                    incl = plsc.cumsum(mi)
                    pos = b16c + incl - mi
                    plsc.store_scatter(vlA, [pos], v16, mask=m)
                    bpos = iota + (c * XC + j * LANES)
                    plsc.store_scatter(blA, [pos], bpos, mask=m)
                    return b16c + _bcast(incl, 15)

                base16 = lax.fori_loop(0, 16, scan_vreg, base16)
                return (base16, collecting, cutoff)

            return lax.fori_loop(0, VPC // 16, scan_block, carry)

        base16, _, cutoff = lax.fori_loop(
            0, NCHUNK, scan_chunk,
            (jnp.zeros((LANES,), jnp.int32), jnp.bool_(True),
             jnp.int32(NVREG)))
        npairs = base16[0]

        st_a[0] = jnp.int32(0)
        ct_a[0] = npairs
        srcs = [(vlA, blA, st_a, ct_a), (vlB, blB, st_b, ct_b)]
        for lvl in range(6):
            vs, bs, sts, cts = srcs[lvl % 2]
            vd, bd, std, ctd = srcs[(lvl + 1) % 2]
            bit = 14 - lvl

            def split_list(i, _, vs=vs, bs=bs, sts=sts, cts=cts,
                           vd=vd, bd=bd, std=std, ctd=ctd, bit=bit):
                s = sts[i]
                c = cts[i]
                nv = (c + LANES - 1) >> 4

                def place_vreg(k, offs):
                    lo16, hi16 = offs
                    v16 = vs[pl.ds(s + k * LANES, LANES)]
                    b16 = bs[pl.ds(s + k * LANES, LANES)]
                    lanem = iota < (c - k * LANES)
                    bitm = ((v16 - alo) >> bit) & 1
                    mlo = jnp.logical_and(lanem, bitm == 0)
                    mhi = jnp.logical_and(lanem, bitm == 1)
                    ilo = plsc.cumsum(mlo.astype(jnp.int32))
                    ihi = plsc.cumsum(mhi.astype(jnp.int32))
                    plo = lo16 + ilo - mlo.astype(jnp.int32)
                    phi = hi16 - ihi
                    plsc.store_scatter(vd, [plo], v16, mask=mlo)
                    plsc.store_scatter(bd, [plo], b16, mask=mlo)
                    plsc.store_scatter(vd, [phi], v16, mask=mhi)
                    plsc.store_scatter(bd, [phi], b16, mask=mhi)
                    return (lo16 + _bcast(ilo, 15), hi16 - _bcast(ihi, 15))

                lo16, _ = lax.fori_loop(
                    0, nv, place_vreg,
                    (jnp.full((LANES,), s, jnp.int32),
                     jnp.full((LANES,), s + c, jnp.int32)))
                bnd = lo16[0]
                std[2 * i] = s
                ctd[2 * i] = bnd - s
                std[2 * i + 1] = bnd
                ctd[2 * i + 1] = s + c - bnd
                return 0

            lax.fori_loop(0, 1 << lvl, split_list, 0)

        def extract_from(u, srow, fired, tail):
            s = st_a[u]
            c = ct_a[u]
            c0 = col0_of(u)
            nv = (c + LANES - 1) >> 4

            def pairs_vreg(k, fired2):
                v16 = vlA[pl.ds(s + k * LANES, LANES)]
                b16 = blA[pl.ds(s + k * LANES, LANES)]
                cmax = D - 1 if tail else SLABW - 1
                c16 = jnp.clip(v16 - c0, 0, cmax)
                sc16 = jnp.where(v16 == 0, 0.0, SCALE).astype(jnp.float32)
                rem = c - k * LANES
                lanem = iota < rem
                bsafe = jnp.where(lanem, b16, B)

                @pl.when(fired2 >= RING)
                def _():
                    drain_out_one()
                rb = (fired2 & (RING - 1)) * LANES
                for r in range(LANES):
                    ccv = _bcast(c16, r)
                    ssv = _bcast(sc16, r)
                    for g in range(D // LANES):
                        dv = iota + (srow + g * LANES)
                        if tail:
                            g16 = plsc.load_gather(tailbuf, [ccv, dv - srow])
                        else:
                            g16 = plsc.load_gather(slab, [dv, ccv])
                        rowbuf[rb + r, pl.ds(g * LANES, LANES)] = g16 * ssv
                pltpu.async_copy(rowbuf.at[pl.ds(rb, LANES)],
                                 out_hbm.at[bsafe], sem_o)
                return fired2 + 1

            return lax.fori_loop(0, nv, pairs_vreg, fired)

        def extract(u, srow, fired):
            return lax.cond(
                is_short_slab(u),
                lambda f: extract_from(u, srow, f, True),
                lambda f: extract_from(u, srow, f, False),
                fired)

        def sweep_pair(t, fired):
            u0 = 2 * t
            drain_slab(u0, 0, sem_a)
            fired = extract(u0, 0, fired)

            @pl.when(u0 + 2 < NSLAB)
            def _():
                fire_slab(u0 + 2, 0, sem_a)
            drain_slab(u0 + 1, D, sem_b)
            fired = extract(u0 + 1, D, fired)

            @pl.when(u0 + 3 < NSLAB)
            def _():
                fire_slab(u0 + 3, D, sem_b)
            return fired

        fired = lax.fori_loop(0, NSLAB // 2, sweep_pair, jnp.int32(0))
        lax.fori_loop(0, jnp.minimum(fired, RING),
                      lambda q, c3: (drain_out_one(), c3)[1], 0)
        return cutoff

    lax.while_loop(lambda s0: s0 < NVREG, round_body, jnp.int32(0))


def kernel(x, table):
    x1 = x.reshape(B).astype(jnp.int32)
    tT = table.T
    tail = jnp.pad(table[V - D:, :], ((0, 0), (0, OW - D)))
    mesh = plsc.VectorSubcoreMesh(core_axis_name="c", subcore_axis_name="s")
    out = pl.kernel(
        _body,
        mesh=mesh,
        compiler_params=pltpu.CompilerParams(needs_layout_passes=False),
        out_type=jax.ShapeDtypeStruct((B + LANES, OW), jnp.float32),
        scratch_types=[
            pltpu.VMEM((2 * XC,), jnp.int32),
            pltpu.VMEM((PC,), jnp.int32),
            pltpu.VMEM((PC,), jnp.int32),
            pltpu.VMEM((PC,), jnp.int32),
            pltpu.VMEM((PC,), jnp.int32),
            pltpu.VMEM((2 * D, SLABW), jnp.float32),
            pltpu.VMEM((RING * LANES, OW), jnp.float32),
            pltpu.VMEM((D, OW), jnp.float32),
            pltpu.SMEM((64,), jnp.int32),
            pltpu.SMEM((64,), jnp.int32),
            pltpu.SMEM((64,), jnp.int32),
            pltpu.SMEM((64,), jnp.int32),
            pltpu.SemaphoreType.DMA,
            pltpu.SemaphoreType.DMA,
            pltpu.SemaphoreType.DMA,
            pltpu.SemaphoreType.DMA,
        ],
    )(x1, tT, tail)
    return out[:B, :D].reshape(x.shape[0], x.shape[1], D)


# speedup vs baseline: 1.6203x; 1.4654x over previous
import jax
import jax.numpy as jnp
from jax import lax
from jax.experimental import pallas as pl
from jax.experimental.pallas import tpu as pltpu
from jax.experimental.pallas import tpu_sc as plsc

D_MODEL = 64
LANES = 16
NUM_CORES = 2
NW = 32
BATCH = 1024 * 200
ROWS_PER_W = BATCH // NW
GROUP = 128
NGROUPS = ROWS_PER_W // GROUP
SCALE = 8.0
PAD = 0

_GATHER_DNUMS = lax.GatherDimensionNumbers(
    offset_dims=(), collapsed_slice_dims=(0,), start_index_map=(0,))


def _emb_body(idx_hbm, table_hbm, out_hbm, idx_v, buf, sem_a, sem_b):
    w = lax.axis_index("s") * NUM_CORES + lax.axis_index("c")
    pltpu.sync_copy(idx_hbm.at[pl.ds(w * ROWS_PER_W, ROWS_PER_W)], idx_v)

    def fire_group(g, half, sem):
        base = g * GROUP

        def fire_body(j, c2):
            idx16 = idx_v[pl.ds(base + j * LANES, LANES)]
            for r in range(LANES):
                v = idx16[r]
                pltpu.async_copy(table_hbm.at[v],
                                 buf.at[half * GROUP + j * LANES + r], sem)
            return c2

        lax.fori_loop(0, GROUP // LANES, fire_body, 0)

    def drain_group(half, sem):
        pltpu.make_async_copy(table_hbm.at[pl.ds(0, GROUP)],
                              buf.at[pl.ds(half * GROUP, GROUP)], sem).wait()

    def scale_out(g, half):
        base = g * GROUP

        def j_body(j, c2):
            idx16 = idx_v[pl.ds(base + j * LANES, LANES)]
            s16 = jnp.where(idx16 == PAD, 0.0, SCALE).astype(jnp.float32)
            for r in range(LANES):
                sv = lax.gather(s16, jnp.full((LANES, 1), r, jnp.int32),
                                _GATHER_DNUMS, (1,),
                                mode=lax.GatherScatterMode.PROMISE_IN_BOUNDS)
                row = half * GROUP + j * LANES + r
                for c in range(D_MODEL // LANES):
                    sl = pl.ds(c * LANES, LANES)
                    buf[row, sl] = buf[row, sl] * sv
            return c2

        lax.fori_loop(0, GROUP // LANES, j_body, 0)
        pltpu.sync_copy(buf.at[pl.ds(half * GROUP, GROUP)],
                        out_hbm.at[pl.ds(w * ROWS_PER_W + base, GROUP)])

    fire_group(0, 0, sem_a)

    def pair_body(t, carry):
        g0 = 2 * t
        fire_group(g0 + 1, 1, sem_b)
        drain_group(0, sem_a)
        scale_out(g0, 0)

        @pl.when(g0 + 2 < NGROUPS)
        def _():
            fire_group(g0 + 2, 0, sem_a)
        drain_group(1, sem_b)
        scale_out(g0 + 1, 1)
        return carry

    lax.fori_loop(0, NGROUPS // 2, pair_body, 0)


def kernel(x, table):
    idx1 = x.reshape(BATCH).astype(jnp.int32)
    mesh = plsc.VectorSubcoreMesh(core_axis_name="c", subcore_axis_name="s")
    out = pl.kernel(
        _emb_body,
        mesh=mesh,
        compiler_params=pltpu.CompilerParams(use_tc_tiling_on_sc=False),
        out_type=jax.ShapeDtypeStruct((BATCH, D_MODEL), jnp.float32),
        scratch_types=[
            pltpu.VMEM((ROWS_PER_W,), jnp.int32),
            pltpu.VMEM((2 * GROUP, D_MODEL), jnp.float32),
            pltpu.SemaphoreType.DMA,
            pltpu.SemaphoreType.DMA,
        ],
    )(idx1, table)
    return out.reshape(x.shape[0], x.shape[1], D_MODEL)


# speedup vs baseline: 1.6242x; 1.0024x over previous
import jax
import jax.numpy as jnp
from jax import lax
from jax.experimental import pallas as pl
from jax.experimental.pallas import tpu as pltpu
from jax.experimental.pallas import tpu_sc as plsc

D_MODEL = 64
LANES = 16
NUM_CORES = 2
NW = 32
BATCH = 1024 * 200
ROWS_PER_W = BATCH // NW
GROUP = 128
NGROUPS = ROWS_PER_W // GROUP
SCALE = 8.0
PAD = 0

_GATHER_DNUMS = lax.GatherDimensionNumbers(
    offset_dims=(), collapsed_slice_dims=(0,), start_index_map=(0,))


def _emb_body(idx_hbm, table_hbm, out_hbm, idx_v, buf, sem_a, sem_b):
    w = lax.axis_index("s") * NUM_CORES + lax.axis_index("c")
    pltpu.sync_copy(idx_hbm.at[pl.ds(w * ROWS_PER_W, ROWS_PER_W)], idx_v)

    def fire_group(g, half, sem):
        base = g * GROUP

        def fire_body(j, c2):
            idx16 = idx_v[pl.ds(base + j * LANES, LANES)]
            for r in range(LANES):
                v = idx16[r]
                pltpu.async_copy(table_hbm.at[v],
                                 buf.at[half * GROUP + j * LANES + r], sem)
            return c2

        lax.fori_loop(0, GROUP // LANES, fire_body, 0)

    def drain_group(half, sem):
        pltpu.make_async_copy(table_hbm.at[pl.ds(0, GROUP)],
                              buf.at[pl.ds(half * GROUP, GROUP)], sem).wait()

    def scale_out(g, half):
        base = g * GROUP

        def j_body(j, c2):
            idx16 = idx_v[pl.ds(base + j * LANES, LANES)]
            s16 = jnp.where(idx16 == PAD, 0.0, SCALE).astype(jnp.float32)
            for r in range(LANES):
                sv = lax.broadcast_in_dim(s16[r], (LANES,), ())
                row = half * GROUP + j * LANES + r
                for c in range(D_MODEL // LANES):
                    sl = pl.ds(c * LANES, LANES)
                    buf[row, sl] = buf[row, sl] * sv
            return c2

        lax.fori_loop(0, GROUP // LANES, j_body, 0)
        pltpu.sync_copy(buf.at[pl.ds(half * GROUP, GROUP)],
                        out_hbm.at[pl.ds(w * ROWS_PER_W + base, GROUP)])

    fire_group(0, 0, sem_a)

    def pair_body(t, carry):
        g0 = 2 * t
        fire_group(g0 + 1, 1, sem_b)
        drain_group(0, sem_a)
        scale_out(g0, 0)

        @pl.when(g0 + 2 < NGROUPS)
        def _():
            fire_group(g0 + 2, 0, sem_a)
        drain_group(1, sem_b)
        scale_out(g0 + 1, 1)
        return carry

    lax.fori_loop(0, NGROUPS // 2, pair_body, 0)


def kernel(x, table):
    idx1 = x.reshape(BATCH).astype(jnp.int32)
    mesh = plsc.VectorSubcoreMesh(core_axis_name="c", subcore_axis_name="s")
    out = pl.kernel(
        _emb_body,
        mesh=mesh,
        compiler_params=pltpu.CompilerParams(use_tc_tiling_on_sc=False),
        out_type=jax.ShapeDtypeStruct((BATCH, D_MODEL), jnp.float32),
        scratch_types=[
            pltpu.VMEM((ROWS_PER_W,), jnp.int32),
            pltpu.VMEM((2 * GROUP, D_MODEL), jnp.float32),
            pltpu.SemaphoreType.DMA,
            pltpu.SemaphoreType.DMA,
        ],
    )(idx1, table)
    return out.reshape(x.shape[0], x.shape[1], D_MODEL)


# speedup vs baseline: 2.7738x; 1.7078x over previous
import jax
import jax.numpy as jnp
from jax import lax
from jax.experimental import pallas as pl
from jax.experimental.pallas import tpu as pltpu
from jax.experimental.pallas import tpu_sc as plsc

D_MODEL = 64
LANES = 16
NUM_CORES = 2
NW = 32
BATCH = 1024 * 200
ROWS_PER_W = BATCH // NW
GROUP = 256
NGROUPS = ROWS_PER_W // GROUP
SCALE = 8.0
PAD = 0


def _emb_body(idx_hbm, table_hbm, out_hbm, idx_v, buf, sem):
    w = lax.axis_index("s") * NUM_CORES + lax.axis_index("c")
    pltpu.sync_copy(idx_hbm.at[pl.ds(w * ROWS_PER_W, ROWS_PER_W)], idx_v)

    def group_body(g, carry):
        base = g * GROUP

        def fire_body(j, c2):
            idx16 = idx_v[pl.ds(base + j * LANES, LANES)]
            for r in range(LANES):
                v = idx16[r]
                pltpu.async_copy(table_hbm.at[v], buf.at[j * LANES + r], sem)
            return c2

        lax.fori_loop(0, GROUP // LANES, fire_body, 0)
        pltpu.make_async_copy(table_hbm.at[pl.ds(0, GROUP)], buf, sem).wait()

        def j_body(j, c2):
            idx16 = idx_v[pl.ds(base + j * LANES, LANES)]
            s16 = jnp.where(idx16 == PAD, 0.0, SCALE).astype(jnp.float32)
            for r in range(LANES):
                sv = lax.broadcast_in_dim(s16[r], (LANES,), ())
                row = j * LANES + r
                for c in range(D_MODEL // LANES):
                    sl = pl.ds(c * LANES, LANES)
                    buf[row, sl] = buf[row, sl] * sv
            return c2

        lax.fori_loop(0, GROUP // LANES, j_body, 0)

        pltpu.sync_copy(buf, out_hbm.at[pl.ds(w * ROWS_PER_W + base, GROUP)])
        return carry

    lax.fori_loop(0, NGROUPS, group_body, 0)


def kernel(x, table):
    idx1 = x.reshape(BATCH).astype(jnp.int32)
    mesh = plsc.VectorSubcoreMesh(core_axis_name="c", subcore_axis_name="s")
    out = pl.kernel(
        _emb_body,
        mesh=mesh,
        out_type=jax.ShapeDtypeStruct((BATCH, D_MODEL), jnp.float32),
        scratch_types=[
            pltpu.VMEM((ROWS_PER_W,), jnp.int32),
            pltpu.VMEM((GROUP, D_MODEL), jnp.float32),
            pltpu.SemaphoreType.DMA,
        ],
    )(idx1, table)
    return out.reshape(x.shape[0], x.shape[1], D_MODEL)


# speedup vs baseline: 2.8366x; 1.0227x over previous
import jax
import jax.numpy as jnp
from jax import lax
from jax.experimental import pallas as pl
from jax.experimental.pallas import tpu as pltpu
from jax.experimental.pallas import tpu_sc as plsc

D_MODEL = 64
LANES = 16
NUM_CORES = 2
NW = 32
BATCH = 1024 * 200
ROWS_PER_W = BATCH // NW
GROUP = 640
NGROUPS = ROWS_PER_W // GROUP
SCALE = 8.0
PAD = 0


def _emb_body(idx_hbm, table_hbm, out_hbm, idx_v, buf, sem):
    w = lax.axis_index("s") * NUM_CORES + lax.axis_index("c")
    pltpu.sync_copy(idx_hbm.at[pl.ds(w * ROWS_PER_W, ROWS_PER_W)], idx_v)

    def group_body(g, carry):
        base = g * GROUP

        def fire_body(j, c2):
            idx16 = idx_v[pl.ds(base + j * LANES, LANES)]
            for r in range(LANES):
                v = idx16[r]
                pltpu.async_copy(table_hbm.at[v], buf.at[j * LANES + r], sem)
            return c2

        lax.fori_loop(0, GROUP // LANES, fire_body, 0)
        pltpu.make_async_copy(table_hbm.at[pl.ds(0, GROUP)], buf, sem).wait()

        def j_body(j, c2):
            idx16 = idx_v[pl.ds(base + j * LANES, LANES)]
            s16 = jnp.where(idx16 == PAD, 0.0, SCALE).astype(jnp.float32)
            for r in range(LANES):
                sv = lax.broadcast_in_dim(s16[r], (LANES,), ())
                row = j * LANES + r
                for c in range(D_MODEL // LANES):
                    sl = pl.ds(c * LANES, LANES)
                    buf[row, sl] = buf[row, sl] * sv
            return c2

        lax.fori_loop(0, GROUP // LANES, j_body, 0)

        pltpu.sync_copy(buf, out_hbm.at[pl.ds(w * ROWS_PER_W + base, GROUP)])
        return carry

    lax.fori_loop(0, NGROUPS, group_body, 0)


def kernel(x, table):
    idx1 = x.reshape(BATCH).astype(jnp.int32)
    mesh = plsc.VectorSubcoreMesh(core_axis_name="c", subcore_axis_name="s")
    out = pl.kernel(
        _emb_body,
        mesh=mesh,
        out_type=jax.ShapeDtypeStruct((BATCH, D_MODEL), jnp.float32),
        scratch_types=[
            pltpu.VMEM((ROWS_PER_W,), jnp.int32),
            pltpu.VMEM((GROUP, D_MODEL), jnp.float32),
            pltpu.SemaphoreType.DMA,
        ],
    )(idx1, table)
    return out.reshape(x.shape[0], x.shape[1], D_MODEL)
